# baseline (speedup 1.0000x reference)
import dataclasses
import functools

import jax
import jax.numpy as jnp
from jax import lax
from jax.experimental import pallas as pl
from jax.experimental.pallas import tpu as pltpu
from jax.experimental.pallas import tpu_sc as plsc

N = 10000
E = 320000
DH = 128

NC = 2
NS = 16
LANES = 16
NW = NC * NS
EPW = E // NW
CH = 40
NBUF = 5
HH = DH // NC
EPT = E // NS
NCHUNK = EPT // CH
RPT = N // NS
ZROWS = 125

@functools.cache
def _vector_mesh():
    return plsc.VectorSubcoreMesh(core_axis_name="c", subcore_axis_name="s",
                                  num_cores=NC, num_subcores=NS)


def _deg_body(ei_hbm, out_hbm, dst_v, degp, sem):
    c = lax.axis_index("c")
    s = lax.axis_index("s")
    wid = s * NC + c
    pltpu.async_copy(ei_hbm.at[1, pl.ds(wid * EPW, EPW)], dst_v, sem).wait()

    zeros = jnp.zeros((LANES,), jnp.float32)

    @pl.loop(0, N, step=LANES)
    def _(i):
        degp[pl.ds(i, LANES)] = zeros

    ones = jnp.ones((LANES,), jnp.float32)

    @pl.loop(0, EPW, step=LANES)
    def _(i):
        idx = dst_v[pl.ds(i, LANES)]
        plsc.addupdate_scatter(degp, [idx], ones)

    pltpu.async_copy(degp, out_hbm.at[wid], sem).wait()


def _sc_compiler_params():
    cp = pltpu.CompilerParams()
    cp = dataclasses.replace(cp, needs_layout_passes=False,
                             use_tc_tiling_on_sc=False)
    return cp


@functools.cache
def _deg_call():
    return pl.kernel(
        _deg_body,
        out_type=jax.ShapeDtypeStruct((NW, N), jnp.float32),
        mesh=_vector_mesh(),
        scratch_types=[
            pltpu.VMEM((EPW,), jnp.int32),
            pltpu.VMEM((N,), jnp.float32),
            pltpu.SemaphoreType.DMA,
        ],
        compiler_params=_sc_compiler_params(),
    )


def _agg_body(g_hbm, ei_hbm, zeros_hbm, out_hbm,
              src_v, dstb, rows_v0, rows_v1, rows_v2, rows_v3, rows_v4,
              gsp_sh, acc_sh, ssem, gsem0, gsem1, gsem2, gsem3, gsem4,
              isem0, isem1, isem2, isem3, isem4):
    c = lax.axis_index("c")
    s = lax.axis_index("s")
    base = s * EPT
    rows = (rows_v0, rows_v1, rows_v2, rows_v3, rows_v4)
    gsems = (gsem0, gsem1, gsem2, gsem3, gsem4)
    isems = (isem0, isem1, isem2, isem3, isem4)

    def stage_dst(jj, t):
        pltpu.async_copy(ei_hbm.at[1, pl.ds(base + jj * CH, CH)], dstb.at[t],
                         isems[t])

    def wait_dst(jj, t):
        pltpu.make_async_copy(ei_hbm.at[1, pl.ds(base + jj * CH, CH)],
                              dstb.at[t], isems[t]).wait()

    def gather(jj, t):
        pltpu.async_copy(gsp_sh.at[src_v.at[pl.ds(jj * CH, CH)]], rows[t],
                         gsems[t])

    def wait_gather(jj, t):
        pltpu.make_async_copy(gsp_sh.at[src_v.at[pl.ds(jj * CH, CH)]], rows[t],
                              gsems[t]).wait()

    pltpu.async_copy(ei_hbm.at[0, pl.ds(base, EPT)], src_v, ssem)

    pltpu.sync_copy(g_hbm.at[c].at[pl.ds(s * RPT, RPT)],
                    gsp_sh.at[pl.ds(s * RPT, RPT)])

    @pl.loop(0, RPT // ZROWS)
    def _(k):
        pltpu.sync_copy(zeros_hbm, acc_sh.at[pl.ds(s * RPT + k * ZROWS, ZROWS)])

    for t in range(NBUF):
        stage_dst(t, t)
    pltpu.make_async_copy(ei_hbm.at[0, pl.ds(base, EPT)], src_v, ssem).wait()

    plsc.subcore_barrier()
    for t in range(2):
        gather(t, t)

    @pl.loop(0, NCHUNK, step=NBUF)
    def _(j):
        for t in range(NBUF):
            jj = j + t
            t2 = (t + 2) % NBUF

            @pl.when(jj + 2 < NCHUNK)
            def _():
                gather(jj + 2, t2)

            wait_gather(jj, t)
            wait_dst(jj, t)
            pltpu.sync_copy(rows[t], acc_sh.at[dstb.at[t]], add=True)

            @pl.when(jj + NBUF < NCHUNK)
            def _():
                stage_dst(jj + NBUF, t)

    plsc.subcore_barrier()
    pltpu.sync_copy(acc_sh.at[pl.ds(s * RPT, RPT)],
                    out_hbm.at[c].at[pl.ds(s * RPT, RPT)])


@functools.cache
def _agg_call():
    return pl.kernel(
        _agg_body,
        out_type=jax.ShapeDtypeStruct((NC, N, HH), jnp.float32),
        mesh=_vector_mesh(),
        scratch_types=(
            [pltpu.VMEM((EPT,), jnp.int32),
             pltpu.VMEM((NBUF, CH), jnp.int32)]
            + [pltpu.VMEM((CH, HH), jnp.float32)] * NBUF
            + [pltpu.VMEM_SHARED((N, HH), jnp.float32)] * 2
            + [pltpu.SemaphoreType.DMA] * (2 * NBUF + 1)
        ),
        compiler_params=_sc_compiler_params(),
    )


def _mm_body(x_ref, w_ref, o_ref):
    o_ref[...] = jnp.dot(x_ref[...], w_ref[...],
                         preferred_element_type=jnp.float32)


def _mm(x, w):
    return pl.pallas_call(
        _mm_body,
        out_shape=jax.ShapeDtypeStruct((x.shape[0], w.shape[1]), jnp.float32),
    )(x, w)


def _scale1_body(degp_ref, h_ref, dis_ref, g_ref):
    ones = jnp.ones((NW, 1), jnp.float32)
    deg = lax.dot_general(degp_ref[...], ones, (((0,), (0,)), ((), ())),
                          preferred_element_type=jnp.float32) + 1.0
    dis = lax.rsqrt(deg)
    dis_ref[...] = dis
    g = h_ref[...] * dis
    g_ref[0] = g[:, :HH]
    g_ref[1] = g[:, HH:]


def _scale1(degp, h1p):
    return pl.pallas_call(
        _scale1_body,
        out_shape=(jax.ShapeDtypeStruct((N, 1), jnp.float32),
                   jax.ShapeDtypeStruct((NC, N, HH), jnp.float32)),
    )(degp, h1p)


def _mid_body(agg_ref, h1p_ref, dis_ref, b1_ref, w2_ref, h2p_ref, g2_ref):
    dis = dis_ref[...]
    a = jnp.concatenate([agg_ref[0], agg_ref[1]], axis=1)
    h = a * dis + h1p_ref[...] * (dis * dis) + b1_ref[...]
    h = jnp.maximum(h, 0.0)
    h2p = jnp.dot(h, w2_ref[...], preferred_element_type=jnp.float32)
    h2p_ref[...] = h2p
    g2 = h2p * dis
    g2_ref[0] = g2[:, :HH]
    g2_ref[1] = g2[:, HH:]


def _mid(agg1, h1p, dis, b1, W2):
    return pl.pallas_call(
        _mid_body,
        out_shape=(jax.ShapeDtypeStruct((N, DH), jnp.float32),
                   jax.ShapeDtypeStruct((NC, N, HH), jnp.float32)),
    )(agg1, h1p, dis, b1, W2)


def _final_body(agg_ref, h2p_ref, dis_ref, b2_ref, o_ref):
    dis = dis_ref[...]
    a = jnp.concatenate([agg_ref[0], agg_ref[1]], axis=1)
    o_ref[...] = a * dis + h2p_ref[...] * (dis * dis) + b2_ref[...]


def _final(agg2, h2p, dis, b2):
    return pl.pallas_call(
        _final_body,
        out_shape=jax.ShapeDtypeStruct((N, DH), jnp.float32),
    )(agg2, h2p, dis, b2)


def kernel(x, edge_index, W1, b1, W2, b2):
    zeros_blk = jnp.zeros((ZROWS, HH), jnp.float32)

    degp = _deg_call()(edge_index)
    h1p = _mm(x, W1)
    dis, g1 = _scale1(degp, h1p)
    agg1 = _agg_call()(g1, edge_index, zeros_blk)
    h2p, g2 = _mid(agg1, h1p, dis, b1, W2)
    agg2 = _agg_call()(g2, edge_index, zeros_blk)
    return _final(agg2, h2p, dis, b2)


# speedup vs baseline: 1.4954x; 1.4954x over previous
import dataclasses
import functools

import jax
import jax.numpy as jnp
from jax import lax
from jax.experimental import pallas as pl
from jax.experimental.pallas import tpu as pltpu
from jax.experimental.pallas import tpu_sc as plsc

N = 10000
E = 320000
DH = 128

NC = 2
NS = 16
LANES = 16
NW = NC * NS
EPW = E // NW
CH = 80
NBUF = 3
NCHUNK = EPW // CH
RPT = N // NS
ZROWS = 125

@functools.cache
def _vector_mesh():
    return plsc.VectorSubcoreMesh(core_axis_name="c", subcore_axis_name="s",
                                  num_cores=NC, num_subcores=NS)


def _deg_body(ei_hbm, out_hbm, dst_v, degp, sem):
    c = lax.axis_index("c")
    s = lax.axis_index("s")
    wid = s * NC + c
    pltpu.async_copy(ei_hbm.at[1, pl.ds(wid * EPW, EPW)], dst_v, sem).wait()

    zeros = jnp.zeros((LANES,), jnp.float32)

    @pl.loop(0, N, step=LANES)
    def _(i):
        degp[pl.ds(i, LANES)] = zeros

    ones = jnp.ones((LANES,), jnp.float32)

    @pl.loop(0, EPW, step=LANES)
    def _(i):
        idx = dst_v[pl.ds(i, LANES)]
        plsc.addupdate_scatter(degp, [idx], ones)

    pltpu.async_copy(degp, out_hbm.at[wid], sem).wait()


def _sc_compiler_params():
    cp = pltpu.CompilerParams()
    cp = dataclasses.replace(cp, needs_layout_passes=False,
                             use_tc_tiling_on_sc=False)
    return cp


@functools.cache
def _deg_call():
    return pl.kernel(
        _deg_body,
        out_type=jax.ShapeDtypeStruct((NW, N), jnp.float32),
        mesh=_vector_mesh(),
        scratch_types=[
            pltpu.VMEM((EPW,), jnp.int32),
            pltpu.VMEM((N,), jnp.float32),
            pltpu.SemaphoreType.DMA,
        ],
        compiler_params=_sc_compiler_params(),
    )


def _agg_body(g_hbm, ei_hbm, zeros_hbm, out_hbm,
              src_v, dst_v, rows_v0, rows_v1, rows_v2,
              acc_sh, ssem, gsem0, gsem1, gsem2):
    c = lax.axis_index("c")
    s = lax.axis_index("s")
    wid = s * NC + c
    base = wid * EPW
    rows = (rows_v0, rows_v1, rows_v2)
    gsems = (gsem0, gsem1, gsem2)

    def gather(jj, t):
        pltpu.async_copy(g_hbm.at[src_v.at[pl.ds(jj * CH, CH)]], rows[t],
                         gsems[t])

    def wait_gather(jj, t):
        pltpu.make_async_copy(g_hbm.at[src_v.at[pl.ds(jj * CH, CH)]], rows[t],
                              gsems[t]).wait()

    def scat(jj, t):
        pltpu.sync_copy(rows[t], acc_sh.at[dst_v.at[pl.ds(jj * CH, CH)]],
                        add=True)

    pltpu.async_copy(ei_hbm.at[0, pl.ds(base, EPW)], src_v, ssem)
    pltpu.async_copy(ei_hbm.at[1, pl.ds(base, EPW)], dst_v, ssem)

    @pl.loop(0, RPT // ZROWS)
    def _(k):
        pltpu.sync_copy(zeros_hbm, acc_sh.at[pl.ds(s * RPT + k * ZROWS, ZROWS)])

    pltpu.make_async_copy(ei_hbm.at[0, pl.ds(base, EPW)], src_v, ssem).wait()
    pltpu.make_async_copy(ei_hbm.at[1, pl.ds(base, EPW)], dst_v, ssem).wait()
    for t in range(2):
        gather(t, t)

    plsc.subcore_barrier()

    @pl.loop(0, NCHUNK - 2, step=NBUF)
    def _(j):
        for t in range(NBUF):
            jj = j + t
            gather(jj + 2, (t + 2) % NBUF)
            wait_gather(jj, t)
            scat(jj, t)

    for jj in (NCHUNK - 2, NCHUNK - 1):
        wait_gather(jj, jj % NBUF)
        scat(jj, jj % NBUF)

    plsc.subcore_barrier()
    pltpu.sync_copy(acc_sh.at[pl.ds(s * RPT, RPT)],
                    out_hbm.at[c].at[pl.ds(s * RPT, RPT)])


@functools.cache
def _agg_call():
    return pl.kernel(
        _agg_body,
        out_type=jax.ShapeDtypeStruct((NC, N, DH), jnp.float32),
        mesh=_vector_mesh(),
        scratch_types=(
            [pltpu.VMEM((EPW,), jnp.int32),
             pltpu.VMEM((EPW,), jnp.int32)]
            + [pltpu.VMEM((CH, DH), jnp.float32)] * NBUF
            + [pltpu.VMEM_SHARED((N, DH), jnp.float32)]
            + [pltpu.SemaphoreType.DMA] * (NBUF + 1)
        ),
        compiler_params=_sc_compiler_params(),
    )


def _mm_body(x_ref, w_ref, o_ref):
    o_ref[...] = jnp.dot(x_ref[...], w_ref[...],
                         preferred_element_type=jnp.float32)


def _mm(x, w):
    return pl.pallas_call(
        _mm_body,
        out_shape=jax.ShapeDtypeStruct((x.shape[0], w.shape[1]), jnp.float32),
    )(x, w)


def _scale1_body(degp_ref, h_ref, dis_ref, g_ref):
    ones = jnp.ones((NW, 1), jnp.float32)
    deg = lax.dot_general(degp_ref[...], ones, (((0,), (0,)), ((), ())),
                          preferred_element_type=jnp.float32) + 1.0
    dis = lax.rsqrt(deg)
    dis_ref[...] = dis
    g_ref[...] = h_ref[...] * dis


def _scale1(degp, h1p):
    return pl.pallas_call(
        _scale1_body,
        out_shape=(jax.ShapeDtypeStruct((N, 1), jnp.float32),
                   jax.ShapeDtypeStruct((N, DH), jnp.float32)),
    )(degp, h1p)


def _mid_body(agg_ref, h1p_ref, dis_ref, b1_ref, w2_ref, h2p_ref, g2_ref):
    dis = dis_ref[...]
    a = agg_ref[0] + agg_ref[1]
    h = a * dis + h1p_ref[...] * (dis * dis) + b1_ref[...]
    h = jnp.maximum(h, 0.0)
    h2p = jnp.dot(h, w2_ref[...], preferred_element_type=jnp.float32)
    h2p_ref[...] = h2p
    g2_ref[...] = h2p * dis


def _mid(agg1, h1p, dis, b1, W2):
    return pl.pallas_call(
        _mid_body,
        out_shape=(jax.ShapeDtypeStruct((N, DH), jnp.float32),
                   jax.ShapeDtypeStruct((N, DH), jnp.float32)),
    )(agg1, h1p, dis, b1, W2)


def _final_body(agg_ref, h2p_ref, dis_ref, b2_ref, o_ref):
    dis = dis_ref[...]
    a = agg_ref[0] + agg_ref[1]
    o_ref[...] = a * dis + h2p_ref[...] * (dis * dis) + b2_ref[...]


def _final(agg2, h2p, dis, b2):
    return pl.pallas_call(
        _final_body,
        out_shape=jax.ShapeDtypeStruct((N, DH), jnp.float32),
    )(agg2, h2p, dis, b2)


def kernel(x, edge_index, W1, b1, W2, b2):
    zeros_blk = jnp.zeros((ZROWS, DH), jnp.float32)

    degp = _deg_call()(edge_index)
    h1p = _mm(x, W1)
    dis, g1 = _scale1(degp, h1p)
    agg1 = _agg_call()(g1, edge_index, zeros_blk)
    h2p, g2 = _mid(agg1, h1p, dis, b1, W2)
    agg2 = _agg_call()(g2, edge_index, zeros_blk)
    return _final(agg2, h2p, dis, b2)
